# baseline (speedup 1.0000x reference)
import functools

import jax
import jax.numpy as jnp
from jax.experimental import pallas as pl
from jax.experimental.pallas import tpu as pltpu

_PASS_BITS = (2, 3, 3, 3, 3, 3, 3, 3, 3, 3, 3)


def _kth_threshold(ukey, kf, batch):
    prefix = jnp.zeros((batch, 1), jnp.uint32)
    sh = 32
    for m in _PASS_BITS:
        sh -= m
        jstar = jnp.zeros((batch, 1), jnp.uint32)
        for j in range(1, 1 << m):
            cand = prefix | jnp.uint32(j << sh)
            cnt = jnp.sum(jnp.where(ukey >= cand, 1.0, 0.0), axis=1,
                          keepdims=True)
            jstar += jnp.where(cnt >= kf, jnp.uint32(1), jnp.uint32(0))
        prefix = prefix | jax.lax.shift_left(jstar, jnp.uint32(sh))
    return prefix


def _select_tanh(w, inv_n, u_t, kf, half):
    bits = jax.lax.bitcast_convert_type(w, jnp.uint32)
    ukey = jnp.where(w < 0, ~bits, bits | jnp.uint32(0x80000000))
    prefix = _kth_threshold(ukey, kf, half)
    s = jnp.where(ukey >= prefix, w * inv_n, 0.0)
    a = jnp.tanh(u_t + s)
    nrm = jnp.sqrt(jnp.sum(a * a, axis=1, keepdims=True))
    return a, 1.0 / (nrm + 1e-6)


def _encoder_kernel(x_ref, wi_ref, wr_ref, out_ref, u_ref, *, steps, batch,
                    rec, kk):
    u_ref[:] = jnp.dot(x_ref[:], wi_ref[:], preferred_element_type=jnp.float32)
    wr = wr_ref[:]
    kf = jnp.float32(kk)
    hb = batch // 2

    def step(t, carry):
        a_a, inv_a, a_b, inv_b, w_b = carry
        w_a = jnp.dot(a_a, wr, preferred_element_type=jnp.float32)
        a_b2, inv_b2 = _select_tanh(
            w_b, inv_b, u_ref[pl.ds(t * batch + hb, hb), :], kf, hb)
        w_b2 = jnp.dot(a_b2, wr, preferred_element_type=jnp.float32)
        a_a2, inv_a2 = _select_tanh(
            w_a, inv_a, u_ref[pl.ds(t * batch, hb), :], kf, hb)
        return a_a2, inv_a2, a_b2, inv_b2, w_b2

    init = (jnp.zeros((hb, rec), jnp.float32), jnp.ones((hb, 1), jnp.float32),
            jnp.zeros((hb, rec), jnp.float32), jnp.ones((hb, 1), jnp.float32),
            jnp.zeros((hb, rec), jnp.float32))
    a_a, inv_a, a_b, inv_b, _ = jax.lax.fori_loop(0, steps, step, init,
                                                  unroll=False)
    out_ref[pl.ds(0, hb), :] = a_a * inv_a
    out_ref[pl.ds(hb, hb), :] = a_b * inv_b


def kernel(x, W_input, W_recurrent):
    B, T, H, W, E = x.shape
    R = T * H * W
    rec = W_recurrent.shape[0]
    kk = int(rec * 0.4)
    x2 = jnp.transpose(x.reshape(B, R, E), (1, 0, 2)).reshape(R * B, E)
    return pl.pallas_call(
        functools.partial(_encoder_kernel, steps=R, batch=B, rec=rec, kk=kk),
        out_shape=jax.ShapeDtypeStruct((B, rec), x.dtype),
        scratch_shapes=[pltpu.VMEM((R * B, rec), jnp.float32)],
    )(x2, W_input, W_recurrent)


# speedup vs baseline: 1.0002x; 1.0002x over previous
import functools

import jax
import jax.numpy as jnp
from jax.experimental import pallas as pl
from jax.experimental.pallas import tpu as pltpu

_PASS_BITS = (2, 3, 3, 3, 3, 3, 3, 3, 3, 3, 3)


def _keys(w):
    bits = jax.lax.bitcast_convert_type(w, jnp.uint32)
    return jnp.where(w < 0, ~bits, bits | jnp.uint32(0x80000000))


def _radix_pass(ukey, prefix, sh, m, kf, batch):
    jstar = jnp.zeros((batch, 1), jnp.uint32)
    for j in range(1, 1 << m):
        cand = prefix | jnp.uint32(j << sh)
        cnt = jnp.sum(jnp.where(ukey >= cand, 1.0, 0.0), axis=1,
                      keepdims=True)
        jstar += jnp.where(cnt >= kf, jnp.uint32(1), jnp.uint32(0))
    return prefix | jax.lax.shift_left(jstar, jnp.uint32(sh))


def _select_while_matmul(ukey, a_other, wr_ref, kf, batch, rec):
    n_chunks = rec // 128
    prefix = jnp.zeros((batch, 1), jnp.uint32)
    sh = 32
    parts = None
    for i, m in enumerate(_PASS_BITS):
        if i < n_chunks:
            c = jnp.dot(a_other[:, i * 128:(i + 1) * 128],
                        wr_ref[pl.ds(i * 128, 128), :],
                        preferred_element_type=jnp.float32)
            parts = c if parts is None else parts + c
        sh -= m
        prefix = _radix_pass(ukey, prefix, sh, m, kf, batch)
    return prefix, parts


def _encoder_kernel(x_ref, wi_ref, wr_ref, out_ref, u_ref, *, steps, batch,
                    rec, kk):
    u_ref[:] = jnp.dot(x_ref[:], wi_ref[:], preferred_element_type=jnp.float32)
    kf = jnp.float32(kk)
    hb = batch // 2

    def step(t, carry):
        a_a, inv_a, a_b, inv_b, w_b = carry
        ukey_b = _keys(w_b)
        prefix_b, w_a = _select_while_matmul(ukey_b, a_a, wr_ref, kf, hb, rec)
        s_b = jnp.where(ukey_b >= prefix_b, w_b * inv_b, 0.0)
        a_b2 = jnp.tanh(u_ref[pl.ds(t * batch + hb, hb), :] + s_b)
        inv_b2 = 1.0 / (jnp.sqrt(jnp.sum(a_b2 * a_b2, axis=1,
                                         keepdims=True)) + 1e-6)
        ukey_a = _keys(w_a)
        prefix_a, w_b2 = _select_while_matmul(ukey_a, a_b2, wr_ref, kf, hb,
                                              rec)
        s_a = jnp.where(ukey_a >= prefix_a, w_a * inv_a, 0.0)
        a_a2 = jnp.tanh(u_ref[pl.ds(t * batch, hb), :] + s_a)
        inv_a2 = 1.0 / (jnp.sqrt(jnp.sum(a_a2 * a_a2, axis=1,
                                         keepdims=True)) + 1e-6)
        return a_a2, inv_a2, a_b2, inv_b2, w_b2

    init = (jnp.zeros((hb, rec), jnp.float32), jnp.ones((hb, 1), jnp.float32),
            jnp.zeros((hb, rec), jnp.float32), jnp.ones((hb, 1), jnp.float32),
            jnp.zeros((hb, rec), jnp.float32))
    a_a, inv_a, a_b, inv_b, _ = jax.lax.fori_loop(0, steps, step, init,
                                                  unroll=False)
    out_ref[pl.ds(0, hb), :] = a_a * inv_a
    out_ref[pl.ds(hb, hb), :] = a_b * inv_b


def kernel(x, W_input, W_recurrent):
    B, T, H, W, E = x.shape
    R = T * H * W
    rec = W_recurrent.shape[0]
    kk = int(rec * 0.4)
    x2 = jnp.transpose(x.reshape(B, R, E), (1, 0, 2)).reshape(R * B, E)
    return pl.pallas_call(
        functools.partial(_encoder_kernel, steps=R, batch=B, rec=rec, kk=kk),
        out_shape=jax.ShapeDtypeStruct((B, rec), x.dtype),
        scratch_shapes=[pltpu.VMEM((R * B, rec), jnp.float32)],
    )(x2, W_input, W_recurrent)


# speedup vs baseline: 1.0004x; 1.0002x over previous
import functools

import jax
import jax.numpy as jnp
from jax.experimental import pallas as pl
from jax.experimental.pallas import tpu as pltpu

_PASS_BITS = (2, 3, 3, 3, 3, 3, 3, 3, 3, 3, 3)


def _keys(w):
    bits = jax.lax.bitcast_convert_type(w, jnp.uint32)
    return jnp.where(w < 0, ~bits, bits | jnp.uint32(0x80000000))


def _radix_pass(ukey, prefix, sh, m, kf, batch):
    jstar = jnp.zeros((batch, 1), jnp.uint32)
    for j in range(1, 1 << m):
        cand = prefix | jnp.uint32(j << sh)
        cnt = jnp.sum(jnp.where(ukey >= cand, 1.0, 0.0), axis=1,
                      keepdims=True)
        jstar += jnp.where(cnt >= kf, jnp.uint32(1), jnp.uint32(0))
    return prefix | jax.lax.shift_left(jstar, jnp.uint32(sh))


def _chunked_matmul(a, wr_ref, rec):
    parts = None
    for i in range(rec // 128):
        c = jnp.dot(a[:, i * 128:(i + 1) * 128],
                    wr_ref[pl.ds(i * 128, 128), :],
                    preferred_element_type=jnp.float32)
        parts = c if parts is None else parts + c
    return parts


_PASSES = []
_sh = 32
for _m in _PASS_BITS:
    _sh -= _m
    _PASSES.append((_sh, _m))


def _encoder_kernel(x_ref, wi_ref, wr_ref, out_ref, u_ref, *, steps, batch,
                    rec, kk):
    u_ref[:] = jnp.dot(x_ref[:], wi_ref[:], preferred_element_type=jnp.float32)
    kf = jnp.float32(kk)
    hb = batch // 2
    np_ = len(_PASSES)
    lead = np_ - 5

    def step(t, carry):
        a_a, inv_a, a_b, inv_b, w_b = carry
        ukey_b = _keys(w_b)
        w_a = _chunked_matmul(a_a, wr_ref, rec)
        prefix_b = jnp.zeros((hb, 1), jnp.uint32)
        prefix_a = jnp.zeros((hb, 1), jnp.uint32)
        for p in range(lead):
            sh, m = _PASSES[p]
            prefix_b = _radix_pass(ukey_b, prefix_b, sh, m, kf, hb)
        ukey_a = _keys(w_a)
        for p in range(lead, np_):
            sh, m = _PASSES[p]
            prefix_b = _radix_pass(ukey_b, prefix_b, sh, m, kf, hb)
            sh, m = _PASSES[p - lead]
            prefix_a = _radix_pass(ukey_a, prefix_a, sh, m, kf, hb)
        s_b = jnp.where(ukey_b >= prefix_b, w_b * inv_b, 0.0)
        a_b2 = jnp.tanh(u_ref[pl.ds(t * batch + hb, hb), :] + s_b)
        inv_b2 = 1.0 / (jnp.sqrt(jnp.sum(a_b2 * a_b2, axis=1,
                                         keepdims=True)) + 1e-6)
        w_b2 = _chunked_matmul(a_b2, wr_ref, rec)
        for p in range(np_ - lead, np_):
            sh, m = _PASSES[p]
            prefix_a = _radix_pass(ukey_a, prefix_a, sh, m, kf, hb)
        s_a = jnp.where(ukey_a >= prefix_a, w_a * inv_a, 0.0)
        a_a2 = jnp.tanh(u_ref[pl.ds(t * batch, hb), :] + s_a)
        inv_a2 = 1.0 / (jnp.sqrt(jnp.sum(a_a2 * a_a2, axis=1,
                                         keepdims=True)) + 1e-6)
        return a_a2, inv_a2, a_b2, inv_b2, w_b2

    init = (jnp.zeros((hb, rec), jnp.float32), jnp.ones((hb, 1), jnp.float32),
            jnp.zeros((hb, rec), jnp.float32), jnp.ones((hb, 1), jnp.float32),
            jnp.zeros((hb, rec), jnp.float32))
    a_a, inv_a, a_b, inv_b, _ = jax.lax.fori_loop(0, steps, step, init,
                                                  unroll=False)
    out_ref[pl.ds(0, hb), :] = a_a * inv_a
    out_ref[pl.ds(hb, hb), :] = a_b * inv_b


def kernel(x, W_input, W_recurrent):
    B, T, H, W, E = x.shape
    R = T * H * W
    rec = W_recurrent.shape[0]
    kk = int(rec * 0.4)
    x2 = jnp.transpose(x.reshape(B, R, E), (1, 0, 2)).reshape(R * B, E)
    return pl.pallas_call(
        functools.partial(_encoder_kernel, steps=R, batch=B, rec=rec, kk=kk),
        out_shape=jax.ShapeDtypeStruct((B, rec), x.dtype),
        scratch_shapes=[pltpu.VMEM((R * B, rec), jnp.float32)],
    )(x2, W_input, W_recurrent)
